# Initial kernel scaffold; baseline (speedup 1.0000x reference)
import jax
import jax.numpy as jnp
from jax.experimental import pallas as pl


def kernel(pred_cls, pred_response, pred_bboxes, label_cls, label_response, label_bboxes):
    raise NotImplementedError("write your pallas kernel here")


# speedup vs baseline: 17.9802x; 17.9802x over previous
import jax
import jax.numpy as jnp
from jax.experimental import pallas as pl
from jax.experimental.pallas import tpu as pltpu

L_COORD, L_OBJ, L_NOOBJ = 5.0, 1.0, 0.5


def _body(pc, pr, pb, lc, lr, lb, out_ref):
    b = pl.program_id(0)
    pc_, lc_ = pc[0], lc[0]
    pr_, lr_ = pr[0], lr[0]
    pb_, lb_ = pb[0], lb[0]

    valid = (lr_[0:1] + lr_[1:2] > 0.9).astype(jnp.float32)

    cls_p = jnp.sum(((pc_ - lc_) ** 2) * valid)
    neg = jnp.sum(((pr_ - lr_) ** 2) * (lr_ < 1.0).astype(jnp.float32))

    def corners(o):
        x1 = o[0:1] - o[2:3] * 0.5
        y1 = o[1:2] - o[3:4] * 0.5
        return x1, y1, x1 + o[2:3], y1 + o[3:4]

    def iou(b1, b2):
        lx = jnp.maximum(b1[0], b2[0])
        ly = jnp.maximum(b1[1], b2[1])
        rx = jnp.minimum(b1[2], b2[2])
        ry = jnp.minimum(b1[3], b2[3])
        inter = jnp.maximum(rx - lx, 0.0) * jnp.maximum(ry - ly, 0.0)
        a1 = (b1[2] - b1[0]) * (b1[3] - b1[1])
        a2 = (b2[2] - b2[0]) * (b2[3] - b2[1])
        return inter / (a1 + a2 - inter + 0.0001)

    iou0 = iou(corners(lb_[0:4]), corners(pb_[0:4]))
    iou1 = iou(corners(lb_[4:8]), corners(pb_[4:8]))
    sel = iou1 > iou0
    best_iou = jnp.where(sel, iou1, iou0)
    best_pr = jnp.where(sel, pr_[1:2], pr_[0:1])
    pobj = jnp.sum(((best_pr - best_iou) ** 2) * valid)

    d = (pb_ - lb_) ** 2
    off0 = d[0:1] + d[1:2] + d[2:3] + d[3:4]
    off1 = d[4:5] + d[5:6] + d[6:7] + d[7:8]
    off = jnp.sum(jnp.where(sel, off1, off0) * valid)

    part = jnp.concatenate(
        [jnp.full((1, 128), pobj, jnp.float32),
         jnp.full((1, 128), neg, jnp.float32),
         jnp.full((1, 128), cls_p, jnp.float32),
         jnp.full((1, 128), off, jnp.float32)], axis=0)

    @pl.when(b == 0)
    def _():
        out_ref[...] = jnp.zeros_like(out_ref)

    out_ref[...] += part


def kernel(pred_cls, pred_response, pred_bboxes, label_cls, label_response, label_bboxes):
    B, CLS, H, W = pred_cls.shape
    BB = pred_response.shape[1]
    HW = H * W
    pc = pred_cls.reshape(B, CLS, HW)
    lc = label_cls.reshape(B, CLS, HW)
    pr = pred_response.reshape(B, BB, HW)
    lr = label_response.reshape(B, BB, HW)
    pb = pred_bboxes.reshape(B, BB * 4, HW)
    lb = label_bboxes.reshape(B, BB * 4, HW)

    acc = pl.pallas_call(
        _body,
        grid=(B,),
        in_specs=[
            pl.BlockSpec((1, CLS, HW), lambda b: (b, 0, 0)),
            pl.BlockSpec((1, BB, HW), lambda b: (b, 0, 0)),
            pl.BlockSpec((1, BB * 4, HW), lambda b: (b, 0, 0)),
            pl.BlockSpec((1, CLS, HW), lambda b: (b, 0, 0)),
            pl.BlockSpec((1, BB, HW), lambda b: (b, 0, 0)),
            pl.BlockSpec((1, BB * 4, HW), lambda b: (b, 0, 0)),
        ],
        out_specs=pl.BlockSpec((4, 128), lambda b: (0, 0)),
        out_shape=jax.ShapeDtypeStruct((4, 128), jnp.float32),
    )(pc, pr, pb, lc, lr, lb)

    inv_b = 1.0 / B
    return {"pObj": acc[0, 0] * (inv_b * L_OBJ),
            "nObj": acc[1, 0] * (inv_b * L_NOOBJ),
            "cls": acc[2, 0] * inv_b,
            "offset": acc[3, 0] * (inv_b * L_COORD)}
